# baseline (speedup 1.0000x reference)
import functools

import jax
import numpy as np
import jax.numpy as jnp
from jax import lax
from jax.experimental import pallas as pl
from jax.experimental.pallas import tpu as pltpu
from jax.experimental.pallas import tpu_sc as plsc

N_NODES = 10000
N_PAD = 10240
N_EDGES = 320000
DIM = 128
OUT_DIM = 40
NC = 2
NS = 16
NW = NC * NS
EDGES_PER_W = N_EDGES // NW
CHUNK = 80
NFULL = EDGES_PER_W // CHUNK
ROWS_PER_TILE = N_PAD // NS
NRB = 3
NIB = 6
GRP = 6
NGRP = -(-NFULL // GRP)


def _sc_agg_body(compute_deg, h_hbm, ei_hbm, *refs):
  nib, nrb = NIB, NRB
  if compute_deg:
    p_hbm, d_hbm = refs[0], refs[1]
    refs = refs[2:]
  else:
    p_hbm = refs[0]
    refs = refs[1:]
  idx_s = refs[0:nib]
  didx = refs[nib:2 * nib]
  rows_bf = refs[2 * nib:2 * nib + nrb]
  rows = refs[2 * nib + nrb:2 * nib + 2 * nrb]
  refs = refs[2 * nib + 2 * nrb:]
  if compute_deg:
    ones_v, acc_sh, deg_sh = refs[0], refs[1], refs[2]
    refs = refs[3:]
  else:
    acc_sh = refs[0]
    refs = refs[1:]
  sem_i = refs[0:nib]
  sem_g = refs[nib:nib + nrb]
  sem_s = refs[nib + nrb:nib + 2 * nrb]
  if compute_deg:
    sem_d = refs[nib + 2 * nrb:nib + 3 * nrb]

  cid = lax.axis_index("c")
  sid = lax.axis_index("s")
  base = (sid * NC + cid) * EDGES_PER_W
  z16 = jnp.zeros((16,), jnp.float32)
  ones16 = jnp.ones((16,), jnp.float32)

  def zrow(r, _):
    for j in range(8):
      rows[0][r, pl.ds(j * 16, 16)] = z16
    return 0
  lax.fori_loop(0, CHUNK, zrow, 0)
  zbase = sid * ROWS_PER_TILE
  for t in range(ROWS_PER_TILE // CHUNK):
    pltpu.sync_copy(rows[0], acc_sh.at[pl.ds(zbase + t * CHUNK, CHUNK)])

  if compute_deg:
    for j in range(ROWS_PER_TILE // 16):
      ones_v[pl.ds(j * 16, 16)] = z16
    pltpu.sync_copy(ones_v.at[pl.ds(0, ROWS_PER_TILE)],
                    deg_sh.at[pl.ds(zbase, ROWS_PER_TILE)])
    for j in range(ROWS_PER_TILE // 16):
      ones_v[pl.ds(j * 16, 16)] = ones16

  plsc.subcore_barrier()

  def idx_load_start(g, bi):
    off = base + g * CHUNK
    pltpu.async_copy(ei_hbm.at[pl.ds(off, CHUNK)], idx_s[bi], sem_i[bi])
    pltpu.async_copy(ei_hbm.at[pl.ds(N_EDGES + off, CHUNK)], didx[bi],
                     sem_i[bi])

  def idx_wait(g, bi):
    off = base + g * CHUNK
    pltpu.make_async_copy(
        ei_hbm.at[pl.ds(off, CHUNK)], idx_s[bi], sem_i[bi]).wait()
    pltpu.make_async_copy(
        ei_hbm.at[pl.ds(N_EDGES + off, CHUNK)], didx[bi], sem_i[bi]).wait()

  def gather_start(b, bi):
    pltpu.async_copy(h_hbm.at[didx[bi]], rows_bf[b], sem_g[b])

  def gather_wait(b, bi):
    pltpu.make_async_copy(h_hbm.at[didx[bi]], rows_bf[b], sem_g[b]).wait()

  mhi = jnp.int32(-65536)

  def convert(b):
    def conv_row(r, _):
      for w in range(DIM // 32):
        v = rows_bf[b][r, pl.ds(16 * w, 16)]
        rows[b][r, pl.ds(32 * w, 16)] = lax.bitcast_convert_type(
            v << 16, jnp.float32)
        rows[b][r, pl.ds(32 * w + 16, 16)] = lax.bitcast_convert_type(
            v & mhi, jnp.float32)
      return 0
    lax.fori_loop(0, CHUNK, conv_row, 0)

  def scatter_start(b, bi):
    pltpu.async_copy(rows[b], acc_sh.at[idx_s[bi]], sem_s[b], add=True)
    if compute_deg:
      pltpu.async_copy(ones_v.at[pl.ds(0, CHUNK)], deg_sh.at[idx_s[bi]],
                       sem_d[b], add=True)

  def scatter_wait(b, bi):
    pltpu.make_async_copy(rows[b], acc_sh.at[idx_s[bi]], sem_s[b]).wait()
    if compute_deg:
      pltpu.make_async_copy(ones_v.at[pl.ds(0, CHUNK)], deg_sh.at[idx_s[bi]],
                            sem_d[b]).wait()

  idx_load_start(0, 0)
  idx_load_start(1, 1)
  idx_wait(0, 0)
  gather_start(0, 0)

  def group(go, _):
    for k in range(GRP):
      g = go * GRP + k

      @pl.when((g >= 2) & (g < NFULL + 2))
      def _():
        scatter_wait((k - 2) % NRB, (k - 2) % NIB)

      @pl.when(g + 2 < NFULL)
      def _():
        idx_load_start(g + 2, (k + 2) % NIB)

      @pl.when(g + 1 < NFULL)
      def _():
        idx_wait(g + 1, (k + 1) % NIB)
        gather_start((k + 1) % NRB, (k + 1) % NIB)

      @pl.when(g < NFULL)
      def _():
        gather_wait(k % NRB, k % NIB)
        convert(k % NRB)
        scatter_start(k % NRB, k % NIB)
    return 0
  lax.fori_loop(0, NGRP, group, 0)

  plsc.subcore_barrier()

  pltpu.sync_copy(
      acc_sh.at[pl.ds(zbase, ROWS_PER_TILE)],
      p_hbm.at[cid, pl.ds(zbase, ROWS_PER_TILE)])
  if compute_deg:
    pltpu.sync_copy(deg_sh.at[pl.ds(zbase, ROWS_PER_TILE)],
                    d_hbm.at[cid, pl.ds(zbase, ROWS_PER_TILE)])


def _make_sc_agg(compute_deg):
  mesh = plsc.VectorSubcoreMesh(core_axis_name="c", subcore_axis_name="s")
  out_type = [jax.ShapeDtypeStruct((NC, N_PAD, DIM), jnp.float32)]
  if compute_deg:
    out_type.append(jax.ShapeDtypeStruct((NC, N_PAD), jnp.float32))
  scratch = [pltpu.VMEM((CHUNK,), jnp.int32) for _ in range(2 * NIB)]
  scratch += [pltpu.VMEM((CHUNK, DIM // 2), jnp.int32) for _ in range(NRB)]
  scratch += [pltpu.VMEM((CHUNK, DIM), jnp.float32) for _ in range(NRB)]
  if compute_deg:
    scratch.append(pltpu.VMEM((ROWS_PER_TILE,), jnp.float32))
  scratch.append(pltpu.VMEM_SHARED((N_PAD, DIM), jnp.float32))
  if compute_deg:
    scratch.append(pltpu.VMEM_SHARED((N_PAD,), jnp.float32))
  n_sems = NIB + NRB + NRB + (NRB if compute_deg else 0)
  scratch += [pltpu.SemaphoreType.DMA] * n_sems
  return pl.kernel(
      functools.partial(_sc_agg_body, compute_deg),
      out_type=tuple(out_type) if compute_deg else out_type[0],
      mesh=mesh,
      scratch_types=tuple(scratch),
      compiler_params=pltpu.CompilerParams(use_tc_tiling_on_sc=False),
  )


def _layer_body(x_ref, p0_ref, p1_ref, inv_ref, wt_ref, wb_ref, b_ref,
                o_ref, ob_ref):
  agg = (p0_ref[0] + p1_ref[0]) * inv_ref[...]
  h = (jnp.dot(x_ref[...], wt_ref[...], preferred_element_type=jnp.float32)
       + jnp.dot(agg, wb_ref[...], preferred_element_type=jnp.float32)
       + b_ref[...])
  h = jnp.maximum(h, 0.0)
  o_ref[...] = h
  ob_ref[...] = h.astype(jnp.bfloat16)


def _tail_body(h1_ref, q0_ref, q1_ref, inv_ref, w2t_ref, w2b_ref, b2_ref,
               w3_ref, b3_ref, w4_ref, b4_ref, o_ref):
  agg = (q0_ref[0] + q1_ref[0]) * inv_ref[...]
  h2 = jnp.maximum(
      jnp.dot(h1_ref[...], w2t_ref[...], preferred_element_type=jnp.float32)
      + jnp.dot(agg, w2b_ref[...], preferred_element_type=jnp.float32)
      + b2_ref[...], 0.0)
  h3 = (jnp.dot(h2, w3_ref[...], preferred_element_type=jnp.float32)
        + b3_ref[...])
  lg = (jnp.dot(h3, w4_ref[...], preferred_element_type=jnp.float32)
        + b4_ref[...])
  m = jnp.max(lg, axis=1, keepdims=True)
  s = jnp.log(jnp.sum(jnp.exp(lg - m), axis=1, keepdims=True))
  o_ref[...] = lg - m - s


_ROW_BLK = 1000
_GRID = N_NODES // _ROW_BLK


def _feat_spec():
  return pl.BlockSpec((_ROW_BLK, DIM), lambda i: (i, 0))


def _full_spec(shape):
  return pl.BlockSpec(shape, lambda i: tuple(0 for _ in shape))


_layer1 = pl.pallas_call(
    _layer_body,
    grid=(_GRID,),
    in_specs=[
        _feat_spec(),
        pl.BlockSpec((1, _ROW_BLK, DIM), lambda i: (0, i, 0)),
        pl.BlockSpec((1, _ROW_BLK, DIM), lambda i: (1, i, 0)),
        pl.BlockSpec((_ROW_BLK, 1), lambda i: (i, 0)),
        _full_spec((DIM, DIM)), _full_spec((DIM, DIM)),
        _full_spec((1, DIM)),
    ],
    out_specs=[_feat_spec(), _feat_spec()],
    out_shape=[jax.ShapeDtypeStruct((N_NODES, DIM), jnp.float32),
               jax.ShapeDtypeStruct((N_NODES, DIM), jnp.bfloat16)],
)

_tail = pl.pallas_call(
    _tail_body,
    grid=(_GRID,),
    in_specs=[
        _feat_spec(),
        pl.BlockSpec((1, _ROW_BLK, DIM), lambda i: (0, i, 0)),
        pl.BlockSpec((1, _ROW_BLK, DIM), lambda i: (1, i, 0)),
        pl.BlockSpec((_ROW_BLK, 1), lambda i: (i, 0)),
        _full_spec((DIM, DIM)), _full_spec((DIM, DIM)),
        _full_spec((1, DIM)),
        _full_spec((DIM, DIM)), _full_spec((1, DIM)),
        _full_spec((DIM, OUT_DIM)), _full_spec((1, OUT_DIM)),
    ],
    out_specs=pl.BlockSpec((_ROW_BLK, OUT_DIM), lambda i: (i, 0)),
    out_shape=jax.ShapeDtypeStruct((N_NODES, OUT_DIM), jnp.float32),
)

_sc_agg_deg = _make_sc_agg(True)
_sc_agg = _make_sc_agg(False)


_PERM = np.concatenate(
    [np.concatenate([np.arange(0, 32, 2), np.arange(1, 32, 2)]) + 32 * blk
     for blk in range(DIM // 32)])


@jax.jit
def kernel(x, edge_index, W1, b1, W2, b2, W3, b3, W4, b4):
  ei = edge_index.reshape(2 * N_EDGES)
  xb = jax.lax.bitcast_convert_type(
      x.astype(jnp.bfloat16).reshape(N_NODES, DIM // 2, 2), jnp.int32)
  p, dhist = _sc_agg_deg(xb, ei)
  deg = dhist.sum(axis=0)
  inv = jnp.where(deg > 0, 1.0 / deg, 0.0)[:, None]
  h1, h1b = _layer1(x, p, p, inv, W1[:DIM], W1[DIM:][_PERM],
                    b1.reshape(1, DIM))
  h1b32 = jax.lax.bitcast_convert_type(
      h1b.reshape(N_NODES, DIM // 2, 2), jnp.int32)
  q = _sc_agg(h1b32, ei)
  out = _tail(h1, q, q, inv, W2[:DIM], W2[DIM:][_PERM], b2.reshape(1, DIM),
              W3, b3.reshape(1, DIM), W4, b4.reshape(1, OUT_DIM))
  return out


# speedup vs baseline: 2.0615x; 2.0615x over previous
import functools

import jax
import jax.numpy as jnp
from jax import lax
from jax.experimental import pallas as pl
from jax.experimental.pallas import tpu as pltpu
from jax.experimental.pallas import tpu_sc as plsc

N_NODES = 10000
N_PAD = 10240
N_EDGES = 320000
DIM = 128
OUT_DIM = 40
NC = 2
NS = 16
NW = NC * NS
EDGES_PER_W = N_EDGES // NW
CHUNK = 80
NFULL = EDGES_PER_W // CHUNK
ROWS_PER_TILE = N_PAD // NS
NRB = 4
NIB = 6
GRP = 12
NGRP = -(-NFULL // GRP)


def _sc_agg_body(compute_deg, h_hbm, ei_hbm, *refs):
  nib, nrb = NIB, NRB
  if compute_deg:
    p_hbm, d_hbm = refs[0], refs[1]
    refs = refs[2:]
  else:
    p_hbm = refs[0]
    refs = refs[1:]
  idx_s = refs[0:nib]
  didx = refs[nib:2 * nib]
  rows = refs[2 * nib:2 * nib + nrb]
  refs = refs[2 * nib + nrb:]
  if compute_deg:
    ones_v, acc_sh, deg_sh = refs[0], refs[1], refs[2]
    refs = refs[3:]
  else:
    acc_sh = refs[0]
    refs = refs[1:]
  sem_i = refs[0:nib]
  sem_g = refs[nib:nib + nrb]
  sem_s = refs[nib + nrb:nib + 2 * nrb]
  if compute_deg:
    sem_d = refs[nib + 2 * nrb:nib + 3 * nrb]

  cid = lax.axis_index("c")
  sid = lax.axis_index("s")
  base = (sid * NC + cid) * EDGES_PER_W
  z16 = jnp.zeros((16,), jnp.float32)
  ones16 = jnp.ones((16,), jnp.float32)

  def zrow(r, _):
    for j in range(8):
      rows[0][r, pl.ds(j * 16, 16)] = z16
    return 0
  lax.fori_loop(0, CHUNK, zrow, 0)
  zbase = sid * ROWS_PER_TILE
  for t in range(ROWS_PER_TILE // CHUNK):
    pltpu.sync_copy(rows[0], acc_sh.at[pl.ds(zbase + t * CHUNK, CHUNK)])

  if compute_deg:
    for j in range(ROWS_PER_TILE // 16):
      ones_v[pl.ds(j * 16, 16)] = z16
    pltpu.sync_copy(ones_v.at[pl.ds(0, ROWS_PER_TILE)],
                    deg_sh.at[pl.ds(zbase, ROWS_PER_TILE)])
    for j in range(ROWS_PER_TILE // 16):
      ones_v[pl.ds(j * 16, 16)] = ones16

  plsc.subcore_barrier()

  def idx_load_start(g, bi):
    off = base + g * CHUNK
    pltpu.async_copy(ei_hbm.at[pl.ds(off, CHUNK)], idx_s[bi], sem_i[bi])
    pltpu.async_copy(ei_hbm.at[pl.ds(N_EDGES + off, CHUNK)], didx[bi],
                     sem_i[bi])

  def idx_wait(g, bi):
    off = base + g * CHUNK
    pltpu.make_async_copy(
        ei_hbm.at[pl.ds(off, CHUNK)], idx_s[bi], sem_i[bi]).wait()
    pltpu.make_async_copy(
        ei_hbm.at[pl.ds(N_EDGES + off, CHUNK)], didx[bi], sem_i[bi]).wait()

  def gather_start(b, bi):
    pltpu.async_copy(h_hbm.at[didx[bi]], rows[b], sem_g[b])

  def gather_wait(b, bi):
    pltpu.make_async_copy(h_hbm.at[didx[bi]], rows[b], sem_g[b]).wait()

  def scatter_start(b, bi):
    pltpu.async_copy(rows[b], acc_sh.at[idx_s[bi]], sem_s[b], add=True)
    if compute_deg:
      pltpu.async_copy(ones_v.at[pl.ds(0, CHUNK)], deg_sh.at[idx_s[bi]],
                       sem_d[b], add=True)

  def scatter_wait(b, bi):
    pltpu.make_async_copy(rows[b], acc_sh.at[idx_s[bi]], sem_s[b]).wait()
    if compute_deg:
      pltpu.make_async_copy(ones_v.at[pl.ds(0, CHUNK)], deg_sh.at[idx_s[bi]],
                            sem_d[b]).wait()

  for h in range(3):
    idx_load_start(h, h)
  idx_wait(0, 0)
  gather_start(0, 0)
  idx_wait(1, 1)
  gather_start(1, 1)

  def group(go, _):
    for k in range(GRP):
      g = go * GRP + k

      @pl.when((g >= 2) & (g < NFULL + 2))
      def _():
        scatter_wait((k - 2) % NRB, (k - 2) % NIB)

      @pl.when(g + 3 < NFULL)
      def _():
        idx_load_start(g + 3, (k + 3) % NIB)

      @pl.when(g + 2 < NFULL)
      def _():
        idx_wait(g + 2, (k + 2) % NIB)
        gather_start((k + 2) % NRB, (k + 2) % NIB)

      @pl.when(g < NFULL)
      def _():
        gather_wait(k % NRB, k % NIB)
        scatter_start(k % NRB, k % NIB)
    return 0
  lax.fori_loop(0, NGRP, group, 0)

  plsc.subcore_barrier()

  pltpu.sync_copy(
      acc_sh.at[pl.ds(zbase, ROWS_PER_TILE)],
      p_hbm.at[cid, pl.ds(zbase, ROWS_PER_TILE)])
  if compute_deg:
    pltpu.sync_copy(deg_sh.at[pl.ds(zbase, ROWS_PER_TILE)],
                    d_hbm.at[cid, pl.ds(zbase, ROWS_PER_TILE)])


def _make_sc_agg(compute_deg):
  mesh = plsc.VectorSubcoreMesh(core_axis_name="c", subcore_axis_name="s")
  out_type = [jax.ShapeDtypeStruct((NC, N_PAD, DIM), jnp.float32)]
  if compute_deg:
    out_type.append(jax.ShapeDtypeStruct((NC, N_PAD), jnp.float32))
  scratch = [pltpu.VMEM((CHUNK,), jnp.int32) for _ in range(2 * NIB)]
  scratch += [pltpu.VMEM((CHUNK, DIM), jnp.float32) for _ in range(NRB)]
  if compute_deg:
    scratch.append(pltpu.VMEM((ROWS_PER_TILE,), jnp.float32))
  scratch.append(pltpu.VMEM_SHARED((N_PAD, DIM), jnp.float32))
  if compute_deg:
    scratch.append(pltpu.VMEM_SHARED((N_PAD,), jnp.float32))
  n_sems = NIB + NRB + NRB + (NRB if compute_deg else 0)
  scratch += [pltpu.SemaphoreType.DMA] * n_sems
  return pl.kernel(
      functools.partial(_sc_agg_body, compute_deg),
      out_type=tuple(out_type) if compute_deg else out_type[0],
      mesh=mesh,
      scratch_types=tuple(scratch),
  )


def _layer_body(x_ref, p0_ref, p1_ref, inv_ref, w_ref, b_ref, o_ref):
  agg = (p0_ref[0] + p1_ref[0]) * inv_ref[...]
  w = w_ref[...]
  h = (jnp.dot(x_ref[...], w[:DIM], preferred_element_type=jnp.float32)
       + jnp.dot(agg, w[DIM:], preferred_element_type=jnp.float32)
       + b_ref[...])
  o_ref[...] = jnp.maximum(h, 0.0)


def _tail_body(h1_ref, q0_ref, q1_ref, inv_ref, w2_ref, b2_ref, w3_ref,
               b3_ref, w4_ref, b4_ref, o_ref):
  agg = (q0_ref[0] + q1_ref[0]) * inv_ref[...]
  w2 = w2_ref[...]
  h2 = jnp.maximum(
      jnp.dot(h1_ref[...], w2[:DIM], preferred_element_type=jnp.float32)
      + jnp.dot(agg, w2[DIM:], preferred_element_type=jnp.float32)
      + b2_ref[...], 0.0)
  h3 = (jnp.dot(h2, w3_ref[...], preferred_element_type=jnp.float32)
        + b3_ref[...])
  lg = (jnp.dot(h3, w4_ref[...], preferred_element_type=jnp.float32)
        + b4_ref[...])
  m = jnp.max(lg, axis=1, keepdims=True)
  s = jnp.log(jnp.sum(jnp.exp(lg - m), axis=1, keepdims=True))
  o_ref[...] = lg - m - s


_ROW_BLK = 1000
_GRID = N_NODES // _ROW_BLK


def _feat_spec():
  return pl.BlockSpec((_ROW_BLK, DIM), lambda i: (i, 0))


def _full_spec(shape):
  return pl.BlockSpec(shape, lambda i: tuple(0 for _ in shape))


_layer1 = pl.pallas_call(
    _layer_body,
    grid=(_GRID,),
    in_specs=[
        _feat_spec(),
        pl.BlockSpec((1, _ROW_BLK, DIM), lambda i: (0, i, 0)),
        pl.BlockSpec((1, _ROW_BLK, DIM), lambda i: (1, i, 0)),
        pl.BlockSpec((_ROW_BLK, 1), lambda i: (i, 0)),
        _full_spec((2 * DIM, DIM)), _full_spec((1, DIM)),
    ],
    out_specs=_feat_spec(),
    out_shape=jax.ShapeDtypeStruct((N_NODES, DIM), jnp.float32),
)

_tail = pl.pallas_call(
    _tail_body,
    grid=(_GRID,),
    in_specs=[
        _feat_spec(),
        pl.BlockSpec((1, _ROW_BLK, DIM), lambda i: (0, i, 0)),
        pl.BlockSpec((1, _ROW_BLK, DIM), lambda i: (1, i, 0)),
        pl.BlockSpec((_ROW_BLK, 1), lambda i: (i, 0)),
        _full_spec((2 * DIM, DIM)), _full_spec((1, DIM)),
        _full_spec((DIM, DIM)), _full_spec((1, DIM)),
        _full_spec((DIM, OUT_DIM)), _full_spec((1, OUT_DIM)),
    ],
    out_specs=pl.BlockSpec((_ROW_BLK, OUT_DIM), lambda i: (i, 0)),
    out_shape=jax.ShapeDtypeStruct((N_NODES, OUT_DIM), jnp.float32),
)

_sc_agg_deg = _make_sc_agg(True)
_sc_agg = _make_sc_agg(False)


@jax.jit
def kernel(x, edge_index, W1, b1, W2, b2, W3, b3, W4, b4):
  ei = edge_index.reshape(2 * N_EDGES)
  p, dhist = _sc_agg_deg(x, ei)
  deg = dhist.sum(axis=0)
  inv = jnp.where(deg > 0, 1.0 / deg, 0.0)[:, None]
  h1 = _layer1(x, p, p, inv, W1, b1.reshape(1, DIM))
  q = _sc_agg(h1, ei)
  out = _tail(h1, q, q, inv, W2, b2.reshape(1, DIM), W3,
              b3.reshape(1, DIM), W4, b4.reshape(1, OUT_DIM))
  return out


# speedup vs baseline: 2.1046x; 1.0209x over previous
import functools

import jax
import jax.numpy as jnp
from jax import lax
from jax.experimental import pallas as pl
from jax.experimental.pallas import tpu as pltpu
from jax.experimental.pallas import tpu_sc as plsc

N_NODES = 10000
N_PAD = 10240
N_EDGES = 320000
DIM = 128
OUT_DIM = 40
NC = 2
NS = 16
NW = NC * NS
EDGES_PER_W = N_EDGES // NW
CHUNK = 80
NFULL = EDGES_PER_W // CHUNK
ROWS_PER_TILE = N_PAD // NS
NRB = 4
NIB = 6
GRP = 12
NGRP = -(-NFULL // GRP)


def _sc_agg_body(compute_deg, h_hbm, ei_hbm, *refs):
  nib, nrb = NIB, NRB
  if compute_deg:
    p_hbm, d_hbm = refs[0], refs[1]
    refs = refs[2:]
  else:
    p_hbm = refs[0]
    refs = refs[1:]
  idx_s = refs[0:nib]
  didx = refs[nib:2 * nib]
  rows = refs[2 * nib:2 * nib + nrb]
  refs = refs[2 * nib + nrb:]
  if compute_deg:
    ones_v, acc_sh, deg_sh = refs[0], refs[1], refs[2]
    refs = refs[3:]
  else:
    acc_sh = refs[0]
    refs = refs[1:]
  sem_i = refs[0:nib]
  sem_g = refs[nib:nib + nrb]
  sem_s = refs[nib + nrb:nib + 2 * nrb]
  if compute_deg:
    sem_d = refs[nib + 2 * nrb:nib + 3 * nrb]

  cid = lax.axis_index("c")
  sid = lax.axis_index("s")
  base = (sid * NC + cid) * EDGES_PER_W
  z16 = jnp.zeros((16,), jnp.float32)
  ones16 = jnp.ones((16,), jnp.float32)

  def zrow(r, _):
    for j in range(8):
      rows[0][r, pl.ds(j * 16, 16)] = z16
    return 0
  lax.fori_loop(0, CHUNK, zrow, 0)
  zbase = sid * ROWS_PER_TILE
  for t in range(ROWS_PER_TILE // CHUNK):
    pltpu.sync_copy(rows[0], acc_sh.at[pl.ds(zbase + t * CHUNK, CHUNK)])

  if compute_deg:
    for j in range(ROWS_PER_TILE // 16):
      ones_v[pl.ds(j * 16, 16)] = z16
    pltpu.sync_copy(ones_v.at[pl.ds(0, ROWS_PER_TILE)],
                    deg_sh.at[pl.ds(zbase, ROWS_PER_TILE)])
    for j in range(ROWS_PER_TILE // 16):
      ones_v[pl.ds(j * 16, 16)] = ones16

  plsc.subcore_barrier()

  def idx_load_start(g, bi):
    off = base + g * CHUNK
    pltpu.async_copy(ei_hbm.at[pl.ds(off, CHUNK)], idx_s[bi], sem_i[bi])
    pltpu.async_copy(ei_hbm.at[pl.ds(N_EDGES + off, CHUNK)], didx[bi],
                     sem_i[bi])

  def idx_wait(g, bi):
    off = base + g * CHUNK
    pltpu.make_async_copy(
        ei_hbm.at[pl.ds(off, CHUNK)], idx_s[bi], sem_i[bi]).wait()
    pltpu.make_async_copy(
        ei_hbm.at[pl.ds(N_EDGES + off, CHUNK)], didx[bi], sem_i[bi]).wait()

  def gather_start(b, bi):
    pltpu.async_copy(h_hbm.at[didx[bi]], rows[b], sem_g[b])

  def gather_wait(b, bi):
    pltpu.make_async_copy(h_hbm.at[didx[bi]], rows[b], sem_g[b]).wait()

  def scatter_start(b, bi):
    pltpu.async_copy(rows[b], acc_sh.at[idx_s[bi]], sem_s[b], add=True)
    if compute_deg:
      pltpu.async_copy(ones_v.at[pl.ds(0, CHUNK)], deg_sh.at[idx_s[bi]],
                       sem_d[b], add=True)

  def scatter_wait(b, bi):
    pltpu.make_async_copy(rows[b], acc_sh.at[idx_s[bi]], sem_s[b]).wait()
    if compute_deg:
      pltpu.make_async_copy(ones_v.at[pl.ds(0, CHUNK)], deg_sh.at[idx_s[bi]],
                            sem_d[b]).wait()

  for h in range(3):
    idx_load_start(h, h)
  idx_wait(0, 0)
  gather_start(0, 0)
  idx_wait(1, 1)
  gather_start(1, 1)

  def group(go, _):
    for k in range(GRP):
      g = go * GRP + k

      @pl.when((g >= 2) & (g < NFULL + 2))
      def _():
        scatter_wait((k - 2) % NRB, (k - 2) % NIB)

      @pl.when(g + 3 < NFULL)
      def _():
        idx_load_start(g + 3, (k + 3) % NIB)

      @pl.when(g + 2 < NFULL)
      def _():
        idx_wait(g + 2, (k + 2) % NIB)
        gather_start((k + 2) % NRB, (k + 2) % NIB)

      @pl.when(g < NFULL)
      def _():
        gather_wait(k % NRB, k % NIB)
        scatter_start(k % NRB, k % NIB)
    return 0
  lax.fori_loop(0, NGRP, group, 0)

  plsc.subcore_barrier()

  pltpu.sync_copy(
      acc_sh.at[pl.ds(zbase, ROWS_PER_TILE)],
      p_hbm.at[cid, pl.ds(zbase, ROWS_PER_TILE)])
  if compute_deg:
    pltpu.sync_copy(deg_sh.at[pl.ds(zbase, ROWS_PER_TILE)],
                    d_hbm.at[cid, pl.ds(zbase, ROWS_PER_TILE)])


def _make_sc_agg(compute_deg):
  mesh = plsc.VectorSubcoreMesh(core_axis_name="c", subcore_axis_name="s")
  out_type = [jax.ShapeDtypeStruct((NC, N_PAD, DIM), jnp.float32)]
  if compute_deg:
    out_type.append(jax.ShapeDtypeStruct((NC, N_PAD), jnp.float32))
  scratch = [pltpu.VMEM((CHUNK,), jnp.int32) for _ in range(2 * NIB)]
  scratch += [pltpu.VMEM((CHUNK, DIM), jnp.float32) for _ in range(NRB)]
  if compute_deg:
    scratch.append(pltpu.VMEM((ROWS_PER_TILE,), jnp.float32))
  scratch.append(pltpu.VMEM_SHARED((N_PAD, DIM), jnp.float32))
  if compute_deg:
    scratch.append(pltpu.VMEM_SHARED((N_PAD,), jnp.float32))
  n_sems = NIB + NRB + NRB + (NRB if compute_deg else 0)
  scratch += [pltpu.SemaphoreType.DMA] * n_sems
  return pl.kernel(
      functools.partial(_sc_agg_body, compute_deg),
      out_type=tuple(out_type) if compute_deg else out_type[0],
      mesh=mesh,
      scratch_types=tuple(scratch),
  )


def _layer_body(x_ref, p0_ref, p1_ref, inv_ref, w_ref, b_ref, o_ref):
  agg = (p0_ref[0] + p1_ref[0]) * inv_ref[...]
  w = w_ref[...]
  h = (jnp.dot(x_ref[...], w[:DIM], preferred_element_type=jnp.float32)
       + jnp.dot(agg, w[DIM:], preferred_element_type=jnp.float32)
       + b_ref[...])
  o_ref[...] = jnp.maximum(h, 0.0)


def _tail_body(h1_ref, q0_ref, q1_ref, inv_ref, w2_ref, b2_ref, w3_ref,
               b3_ref, w4_ref, b4_ref, o_ref):
  agg = (q0_ref[0] + q1_ref[0]) * inv_ref[...]
  w2 = w2_ref[...]
  h2 = jnp.maximum(
      jnp.dot(h1_ref[...], w2[:DIM], preferred_element_type=jnp.float32)
      + jnp.dot(agg, w2[DIM:], preferred_element_type=jnp.float32)
      + b2_ref[...], 0.0)
  h3 = (jnp.dot(h2, w3_ref[...], preferred_element_type=jnp.float32)
        + b3_ref[...])
  lg = (jnp.dot(h3, w4_ref[...], preferred_element_type=jnp.float32)
        + b4_ref[...])
  m = jnp.max(lg, axis=1, keepdims=True)
  s = jnp.log(jnp.sum(jnp.exp(lg - m), axis=1, keepdims=True))
  o_ref[...] = lg - m - s


_ROW_BLK = 2000
_GRID = N_NODES // _ROW_BLK


def _feat_spec():
  return pl.BlockSpec((_ROW_BLK, DIM), lambda i: (i, 0))


def _full_spec(shape):
  return pl.BlockSpec(shape, lambda i: tuple(0 for _ in shape))


_layer1 = pl.pallas_call(
    _layer_body,
    grid=(_GRID,),
    in_specs=[
        _feat_spec(),
        pl.BlockSpec((1, _ROW_BLK, DIM), lambda i: (0, i, 0)),
        pl.BlockSpec((1, _ROW_BLK, DIM), lambda i: (1, i, 0)),
        pl.BlockSpec((_ROW_BLK, 1), lambda i: (i, 0)),
        _full_spec((2 * DIM, DIM)), _full_spec((1, DIM)),
    ],
    out_specs=_feat_spec(),
    out_shape=jax.ShapeDtypeStruct((N_NODES, DIM), jnp.float32),
)

_tail = pl.pallas_call(
    _tail_body,
    grid=(_GRID,),
    in_specs=[
        _feat_spec(),
        pl.BlockSpec((1, _ROW_BLK, DIM), lambda i: (0, i, 0)),
        pl.BlockSpec((1, _ROW_BLK, DIM), lambda i: (1, i, 0)),
        pl.BlockSpec((_ROW_BLK, 1), lambda i: (i, 0)),
        _full_spec((2 * DIM, DIM)), _full_spec((1, DIM)),
        _full_spec((DIM, DIM)), _full_spec((1, DIM)),
        _full_spec((DIM, OUT_DIM)), _full_spec((1, OUT_DIM)),
    ],
    out_specs=pl.BlockSpec((_ROW_BLK, OUT_DIM), lambda i: (i, 0)),
    out_shape=jax.ShapeDtypeStruct((N_NODES, OUT_DIM), jnp.float32),
)

_sc_agg_deg = _make_sc_agg(True)
_sc_agg = _make_sc_agg(False)


@jax.jit
def kernel(x, edge_index, W1, b1, W2, b2, W3, b3, W4, b4):
  ei = edge_index.reshape(2 * N_EDGES)
  p, dhist = _sc_agg_deg(x, ei)
  deg = dhist.sum(axis=0)
  inv = jnp.where(deg > 0, 1.0 / deg, 0.0)[:, None]
  h1 = _layer1(x, p, p, inv, W1, b1.reshape(1, DIM))
  q = _sc_agg(h1, ei)
  out = _tail(h1, q, q, inv, W2, b2.reshape(1, DIM), W3,
              b3.reshape(1, DIM), W4, b4.reshape(1, OUT_DIM))
  return out
